# Initial kernel scaffold; baseline (speedup 1.0000x reference)
import jax
import jax.numpy as jnp
from jax.experimental import pallas as pl


def kernel(x, emb_table):
    raise NotImplementedError("write your pallas kernel here")


# speedup vs baseline: 1.0056x; 1.0056x over previous
import jax
import jax.numpy as jnp
from jax.experimental import pallas as pl

_F_BLK = 32


def _add_kernel(x_ref, emb_ref, o_ref):
    fe = emb_ref[...].T
    o_ref[...] = x_ref[...] + fe[None, :, :, None]


def kernel(x, emb_table):
    b, c, f, t = x.shape
    grid = (b, f // _F_BLK)
    return pl.pallas_call(
        _add_kernel,
        grid=grid,
        in_specs=[
            pl.BlockSpec((1, c, _F_BLK, t), lambda i, j: (i, 0, j, 0)),
            pl.BlockSpec((_F_BLK, c), lambda i, j: (j, 0)),
        ],
        out_specs=pl.BlockSpec((1, c, _F_BLK, t), lambda i, j: (i, 0, j, 0)),
        out_shape=jax.ShapeDtypeStruct(x.shape, x.dtype),
    )(x, emb_table)
